# baseline (speedup 1.0000x reference)
import functools

import jax
import jax.numpy as jnp
from jax import lax
from jax.experimental import pallas as pl
from jax.experimental.pallas import tpu as pltpu
from jax.experimental.pallas import tpu_sc as plsc

BSZ = 4
NTILE = 4
NTOK = 1601
DIM = 1280
NSLAB = BSZ * NTILE
T = 16
NCH_SUB = 50
NCOL = DIM // 16


def _body(x_hbm, arh_hbm, arw_hbm, gate_hbm, emb_hbm, out_hbm,
          arh_v, arw_v, gate_v, idx_v, rows_v, pos_v, in_v, out_v,
          sin0, sin1, sout0, sout1):
  cid = lax.axis_index("c")
  sid = lax.axis_index("s")
  wid = sid * 2 + cid
  slab = wid // 2
  half = wid % 2

  pltpu.sync_copy(arh_hbm, arh_v)
  pltpu.sync_copy(arw_hbm, arw_v)
  pltpu.sync_copy(gate_hbm, gate_v)

  lanes = lax.iota(jnp.int32, 16)
  four = jnp.full((16,), NTILE, jnp.int32)
  one16 = jnp.full((16,), 1, jnp.int32)
  t_vec = lax.rem(lanes, four)
  h_vec = arh_v[...]
  w_vec = arw_v[...]
  w_safe = jnp.maximum(w_vec, one16)
  row = lax.div(t_vec, w_safe)
  col = lax.rem(t_vec, w_safe)
  m = jnp.minimum(jnp.maximum(h_vec * w_vec - t_vec, one16 - one16), one16)
  emb_idx = m * (row * four + col) + (one16 - m) * jnp.full(
      (16,), NSLAB, jnp.int32)
  idx_v[...] = emb_idx

  gcp = pltpu.make_async_copy(emb_hbm.at[idx_v], rows_v, sin0)
  gcp.start()
  gcp.wait()

  g = gate_v[...]
  tanh_g = 1.0 - 2.0 / (jnp.exp(2.0 * g) + 1.0)

  def build(j, _):
    pos_v[pl.ds(j * 16, 16)] = rows_v[slab, pl.ds(j * 16, 16)] * tanh_g
    return 0
  lax.fori_loop(0, NCOL, build, 0)

  def chunk_start(c):
    return (half * NCH_SUB + c) * T

  in_sems = (sin0, sin1)
  out_sems = (sout0, sout1)

  def start_in(c, buf):
    s = chunk_start(c)
    pltpu.make_async_copy(x_hbm.at[slab, pl.ds(s, T)], in_v.at[buf],
                          in_sems[buf]).start()

  def wait_in(buf):
    pltpu.make_async_copy(x_hbm.at[slab, pl.ds(0, T)], in_v.at[buf],
                          in_sems[buf]).wait()

  def start_out(c, buf):
    s = chunk_start(c)
    pltpu.make_async_copy(out_v.at[buf], out_hbm.at[slab, pl.ds(s, T)],
                          out_sems[buf]).start()

  def wait_out(buf):
    pltpu.make_async_copy(out_v.at[buf], out_hbm.at[slab, pl.ds(0, T)],
                          out_sems[buf]).wait()

  def compute(buf):
    def col_body(j, _):
      off = j * 16
      pos_j = pos_v[pl.ds(off, 16)]
      def tok_body(t, _):
        out_v[buf, t, pl.ds(off, 16)] = in_v[buf, t, pl.ds(off, 16)] + pos_j
        return 0
      lax.fori_loop(0, T, tok_body, 0)
      return 0
    lax.fori_loop(0, NCOL, col_body, 0)

  start_in(0, 0)
  start_in(1, 1)

  def group(gi, _):
    c0 = gi * 2
    for buf in (0, 1):
      c = c0 + buf
      wait_in(buf)

      @pl.when(gi > 0)
      def _():
        wait_out(buf)

      compute(buf)
      start_out(c, buf)

      @pl.when(c + 2 < NCH_SUB)
      def _():
        start_in(c + 2, buf)
    return 0

  lax.fori_loop(0, NCH_SUB // 2, group, 0)
  wait_out(0)
  wait_out(1)

  @pl.when(half == 0)
  def _():
    pltpu.make_async_copy(x_hbm.at[slab, pl.ds(1600, 1)], in_v.at[0, pl.ds(0, 1)],
                          sin0).start()
    pltpu.make_async_copy(x_hbm.at[slab, pl.ds(1600, 1)], in_v.at[0, pl.ds(0, 1)],
                          sin0).wait()

    def lr_body(j, _):
      off = j * 16
      out_v[0, 0, pl.ds(off, 16)] = in_v[0, 0, pl.ds(off, 16)] + pos_v[pl.ds(off, 16)]
      return 0
    lax.fori_loop(0, NCOL, lr_body, 0)
    pltpu.make_async_copy(out_v.at[0, pl.ds(0, 1)], out_hbm.at[slab, pl.ds(1600, 1)],
                          sout0).start()
    pltpu.make_async_copy(out_v.at[0, pl.ds(0, 1)], out_hbm.at[slab, pl.ds(1600, 1)],
                          sout0).wait()


@jax.jit
def kernel(x, aspect_ratio, embedding, gate):
  x3 = x.reshape(NSLAB, NTOK, DIM)
  ar32 = aspect_ratio.astype(jnp.int32)
  arh16 = jnp.repeat(ar32[:, 0], NTILE)
  arw16 = jnp.repeat(ar32[:, 1], NTILE)
  gate16 = jnp.broadcast_to(gate.astype(jnp.float32), (16,))
  emb2 = jnp.concatenate(
      [embedding.reshape(NSLAB, DIM),
       jnp.zeros((1, DIM), jnp.float32)])

  mesh = plsc.VectorSubcoreMesh(core_axis_name="c", subcore_axis_name="s")
  run = functools.partial(
      pl.kernel,
      out_type=jax.ShapeDtypeStruct((NSLAB, NTOK, DIM), jnp.float32),
      mesh=mesh,
      compiler_params=pltpu.CompilerParams(use_tc_tiling_on_sc=True),
      scratch_types=[
          pltpu.VMEM((16,), jnp.int32),
          pltpu.VMEM((16,), jnp.int32),
          pltpu.VMEM((16,), jnp.float32),
          pltpu.VMEM((16,), jnp.int32),
          pltpu.VMEM((NSLAB, DIM), jnp.float32),
          pltpu.VMEM((DIM,), jnp.float32),
          pltpu.VMEM((2, T, DIM), jnp.float32),
          pltpu.VMEM((2, T, DIM), jnp.float32),
          pltpu.SemaphoreType.DMA,
          pltpu.SemaphoreType.DMA,
          pltpu.SemaphoreType.DMA,
          pltpu.SemaphoreType.DMA,
      ],
  )(_body)
  out = run(x3, arh16, arw16, gate16, emb2)
  return out.reshape(BSZ, NTILE, NTOK, DIM)


# speedup vs baseline: 1.1651x; 1.1651x over previous
import functools

import jax
import jax.numpy as jnp
from jax import lax
from jax.experimental import pallas as pl
from jax.experimental.pallas import tpu as pltpu
from jax.experimental.pallas import tpu_sc as plsc

BSZ = 4
NTILE = 4
NTOK = 1601
DIM = 1280
NSLAB = BSZ * NTILE
TB = 416


def _pos_body(arh_hbm, arw_hbm, emb_hbm, pos_hbm, arh_v, arw_v, idx_v,
              rows_v, sem):
  cid = lax.axis_index("c")
  sid = lax.axis_index("s")
  wid = sid * 2 + cid

  @pl.when(wid == 0)
  def _():
    pltpu.sync_copy(arh_hbm, arh_v)
    pltpu.sync_copy(arw_hbm, arw_v)

    lanes = lax.iota(jnp.int32, 16)
    four = jnp.full((16,), NTILE, jnp.int32)
    one16 = jnp.full((16,), 1, jnp.int32)
    t_vec = lax.rem(lanes, four)
    h_vec = arh_v[...]
    w_vec = arw_v[...]
    w_safe = jnp.maximum(w_vec, one16)
    row = lax.div(t_vec, w_safe)
    col = lax.rem(t_vec, w_safe)
    m = jnp.minimum(jnp.maximum(h_vec * w_vec - t_vec, one16 - one16), one16)
    emb_idx = m * (row * four + col) + (one16 - m) * jnp.full(
        (16,), NSLAB, jnp.int32)
    idx_v[...] = emb_idx

    gcp = pltpu.make_async_copy(emb_hbm.at[idx_v], rows_v, sem)
    gcp.start()
    gcp.wait()
    pltpu.sync_copy(rows_v, pos_hbm)


def _sc_pos_table(arh16, arw16, emb2):
  mesh = plsc.VectorSubcoreMesh(core_axis_name="c", subcore_axis_name="s")
  run = functools.partial(
      pl.kernel,
      out_type=jax.ShapeDtypeStruct((NSLAB, DIM), jnp.float32),
      mesh=mesh,
      scratch_types=[
          pltpu.VMEM((16,), jnp.int32),
          pltpu.VMEM((16,), jnp.int32),
          pltpu.VMEM((16,), jnp.int32),
          pltpu.VMEM((NSLAB, DIM), jnp.float32),
          pltpu.SemaphoreType.DMA,
      ],
  )(_pos_body)
  return run(arh16, arw16, emb2)


def _add_body(x_ref, pos_ref, gate_ref, o_ref):
  g = jnp.tanh(gate_ref[0, 0])
  o_ref[...] = x_ref[...] + pos_ref[...] * g


def _tc_add(x3, pos3, gate2):
  grid = (NSLAB, pl.cdiv(NTOK, TB))
  return pl.pallas_call(
      _add_body,
      grid=grid,
      in_specs=[
          pl.BlockSpec((1, TB, DIM), lambda i, j: (i, j, 0)),
          pl.BlockSpec((1, 1, DIM), lambda i, j: (i, 0, 0)),
          pl.BlockSpec(memory_space=pltpu.SMEM),
      ],
      out_specs=pl.BlockSpec((1, TB, DIM), lambda i, j: (i, j, 0)),
      out_shape=jax.ShapeDtypeStruct((NSLAB, NTOK, DIM), jnp.float32),
  )(x3, pos3, gate2)


@jax.jit
def kernel(x, aspect_ratio, embedding, gate):
  x3 = x.reshape(NSLAB, NTOK, DIM)
  ar32 = aspect_ratio.astype(jnp.int32)
  arh16 = jnp.repeat(ar32[:, 0], NTILE)
  arw16 = jnp.repeat(ar32[:, 1], NTILE)
  gate2 = gate.astype(jnp.float32).reshape(1, 1)
  emb2 = jnp.concatenate(
      [embedding.reshape(NSLAB, DIM),
       jnp.zeros((1, DIM), jnp.float32)])

  pos = _sc_pos_table(arh16, arw16, emb2)
  out = _tc_add(x3, pos.reshape(NSLAB, 1, DIM), gate2)
  return out.reshape(BSZ, NTILE, NTOK, DIM)


# speedup vs baseline: 4.1378x; 3.5513x over previous
import functools

import jax
import jax.numpy as jnp
from jax import lax
from jax.experimental import pallas as pl
from jax.experimental.pallas import tpu as pltpu
from jax.experimental.pallas import tpu_sc as plsc

BSZ = 4
NTILE = 4
NTOK = 1601
DIM = 1280
NSLAB = BSZ * NTILE
TB = 416


def _pos_body(arh_hbm, arw_hbm, emb_hbm, pos_hbm, arh_v, arw_v, idx_v,
              rows_v, sem):
  cid = lax.axis_index("c")
  sid = lax.axis_index("s")
  wid = sid * 2 + cid

  @pl.when(wid == 0)
  def _():
    pltpu.sync_copy(arh_hbm, arh_v)
    pltpu.sync_copy(arw_hbm, arw_v)

    lanes = lax.iota(jnp.int32, 16)
    four = jnp.full((16,), NTILE, jnp.int32)
    one16 = jnp.full((16,), 1, jnp.int32)
    t_vec = lax.rem(lanes, four)
    h_vec = arh_v[...]
    w_vec = arw_v[...]
    w_safe = jnp.maximum(w_vec, one16)
    row = lax.div(t_vec, w_safe)
    col = lax.rem(t_vec, w_safe)
    m = jnp.minimum(jnp.maximum(h_vec * w_vec - t_vec, one16 - one16), one16)
    emb_idx = m * (row * four + col) + (one16 - m) * jnp.full(
        (16,), NSLAB, jnp.int32)
    idx_v[...] = emb_idx

    gcp = pltpu.make_async_copy(emb_hbm.at[idx_v], rows_v, sem)
    gcp.start()
    gcp.wait()
    pltpu.sync_copy(rows_v, pos_hbm)


def _sc_pos_table(arh16, arw16, emb2):
  mesh = plsc.VectorSubcoreMesh(core_axis_name="c", subcore_axis_name="s")
  run = functools.partial(
      pl.kernel,
      out_type=jax.ShapeDtypeStruct((NSLAB, DIM), jnp.float32),
      mesh=mesh,
      scratch_types=[
          pltpu.VMEM((16,), jnp.int32),
          pltpu.VMEM((16,), jnp.int32),
          pltpu.VMEM((16,), jnp.int32),
          pltpu.VMEM((NSLAB, DIM), jnp.float32),
          pltpu.SemaphoreType.DMA,
      ],
  )(_pos_body)
  return run(arh16, arw16, emb2)


def _add_body(x_ref, pos_ref, gate_ref, o_ref):
  g = jnp.tanh(gate_ref[0, 0])
  o_ref[...] = x_ref[...] + pos_ref[...] * g


def _tc_add(x, pos4, gate2):
  grid = (BSZ, NTILE, pl.cdiv(NTOK, TB))
  return pl.pallas_call(
      _add_body,
      grid=grid,
      in_specs=[
          pl.BlockSpec((1, 1, TB, DIM), lambda b, t, j: (b, t, j, 0)),
          pl.BlockSpec((1, 1, 1, DIM), lambda b, t, j: (b, t, 0, 0)),
          pl.BlockSpec(memory_space=pltpu.SMEM),
      ],
      out_specs=pl.BlockSpec((1, 1, TB, DIM), lambda b, t, j: (b, t, j, 0)),
      out_shape=jax.ShapeDtypeStruct((BSZ, NTILE, NTOK, DIM), jnp.float32),
  )(x, pos4, gate2)


@jax.jit
def kernel(x, aspect_ratio, embedding, gate):
  ar32 = aspect_ratio.astype(jnp.int32)
  arh16 = jnp.repeat(ar32[:, 0], NTILE)
  arw16 = jnp.repeat(ar32[:, 1], NTILE)
  gate2 = gate.astype(jnp.float32).reshape(1, 1)
  emb2 = jnp.concatenate(
      [embedding.reshape(NSLAB, DIM),
       jnp.zeros((1, DIM), jnp.float32)])

  pos = _sc_pos_table(arh16, arw16, emb2)
  return _tc_add(x, pos.reshape(BSZ, NTILE, 1, DIM), gate2)


# speedup vs baseline: 12.5643x; 3.0364x over previous
import functools

import jax
import jax.numpy as jnp
from jax import lax
from jax.experimental import pallas as pl
from jax.experimental.pallas import tpu as pltpu
from jax.experimental.pallas import tpu_sc as plsc

BSZ = 4
NTILE = 4
NTOK = 1601
DIM = 1280
NSLAB = BSZ * NTILE
TB = 128


def _pos_body(arh_hbm, arw_hbm, emb_hbm, pos_hbm, arh_v, arw_v, idx_v,
              rows_v, sem):
  cid = lax.axis_index("c")
  sid = lax.axis_index("s")
  wid = sid * 2 + cid

  @pl.when(wid == 0)
  def _():
    pltpu.sync_copy(arh_hbm, arh_v)
    pltpu.sync_copy(arw_hbm, arw_v)

    lanes = lax.iota(jnp.int32, 16)
    four = jnp.full((16,), NTILE, jnp.int32)
    one16 = jnp.full((16,), 1, jnp.int32)
    t_vec = lax.rem(lanes, four)
    h_vec = arh_v[...]
    w_vec = arw_v[...]
    w_safe = jnp.maximum(w_vec, one16)
    row = lax.div(t_vec, w_safe)
    col = lax.rem(t_vec, w_safe)
    m = jnp.minimum(jnp.maximum(h_vec * w_vec - t_vec, one16 - one16), one16)
    emb_idx = m * (row * four + col) + (one16 - m) * jnp.full(
        (16,), NSLAB, jnp.int32)
    idx_v[...] = emb_idx

    gcp = pltpu.make_async_copy(emb_hbm.at[idx_v], rows_v, sem)
    gcp.start()
    gcp.wait()
    pltpu.sync_copy(rows_v, pos_hbm)


def _sc_pos_table(arh16, arw16, emb2):
  mesh = plsc.VectorSubcoreMesh(core_axis_name="c", subcore_axis_name="s")
  run = functools.partial(
      pl.kernel,
      out_type=jax.ShapeDtypeStruct((NSLAB, DIM), jnp.float32),
      mesh=mesh,
      scratch_types=[
          pltpu.VMEM((16,), jnp.int32),
          pltpu.VMEM((16,), jnp.int32),
          pltpu.VMEM((16,), jnp.int32),
          pltpu.VMEM((NSLAB, DIM), jnp.float32),
          pltpu.SemaphoreType.DMA,
      ],
  )(_pos_body)
  return run(arh16, arw16, emb2)


def _add_body(x_ref, pos_ref, gate_ref, o_ref):
  g = jnp.tanh(gate_ref[0, 0])
  o_ref[...] = x_ref[...] + pos_ref[...] * g


def _tc_add(xt, pos4, gate2):
  grid = (BSZ, pl.cdiv(NTOK, TB))
  return pl.pallas_call(
      _add_body,
      grid=grid,
      in_specs=[
          pl.BlockSpec((1, TB, NTILE, DIM), lambda b, j: (b, j, 0, 0)),
          pl.BlockSpec((1, 1, NTILE, DIM), lambda b, j: (b, 0, 0, 0)),
          pl.BlockSpec(memory_space=pltpu.SMEM),
      ],
      out_specs=pl.BlockSpec((1, TB, NTILE, DIM), lambda b, j: (b, j, 0, 0)),
      out_shape=jax.ShapeDtypeStruct((BSZ, NTOK, NTILE, DIM), jnp.float32),
  )(xt, pos4, gate2)


@jax.jit
def kernel(x, aspect_ratio, embedding, gate):
  ar32 = aspect_ratio.astype(jnp.int32)
  arh16 = jnp.repeat(ar32[:, 0], NTILE)
  arw16 = jnp.repeat(ar32[:, 1], NTILE)
  gate2 = gate.astype(jnp.float32).reshape(1, 1)
  emb2 = jnp.concatenate(
      [embedding.reshape(NSLAB, DIM),
       jnp.zeros((1, DIM), jnp.float32)])

  pos = _sc_pos_table(arh16, arw16, emb2)
  xt = x.transpose(0, 2, 1, 3)
  out = _tc_add(xt, pos.reshape(BSZ, 1, NTILE, DIM), gate2)
  return out.transpose(0, 2, 1, 3)


# speedup vs baseline: 13.3283x; 1.0608x over previous
import functools

import jax
import jax.numpy as jnp
from jax import lax
from jax.experimental import pallas as pl
from jax.experimental.pallas import tpu as pltpu
from jax.experimental.pallas import tpu_sc as plsc

BSZ = 4
NTILE = 4
NTOK = 1601
DIM = 1280
NSLAB = BSZ * NTILE
TB = 401


def _pos_body(arh_hbm, arw_hbm, emb_hbm, pos_hbm, arh_v, arw_v, idx_v,
              rows_v, sem):
  cid = lax.axis_index("c")
  sid = lax.axis_index("s")
  wid = sid * 2 + cid

  @pl.when(wid == 0)
  def _():
    pltpu.sync_copy(arh_hbm, arh_v)
    pltpu.sync_copy(arw_hbm, arw_v)

    lanes = lax.iota(jnp.int32, 16)
    four = jnp.full((16,), NTILE, jnp.int32)
    one16 = jnp.full((16,), 1, jnp.int32)
    t_vec = lax.rem(lanes, four)
    h_vec = arh_v[...]
    w_vec = arw_v[...]
    w_safe = jnp.maximum(w_vec, one16)
    row = lax.div(t_vec, w_safe)
    col = lax.rem(t_vec, w_safe)
    m = jnp.minimum(jnp.maximum(h_vec * w_vec - t_vec, one16 - one16), one16)
    emb_idx = m * (row * four + col) + (one16 - m) * jnp.full(
        (16,), NSLAB, jnp.int32)
    idx_v[...] = emb_idx

    gcp = pltpu.make_async_copy(emb_hbm.at[idx_v], rows_v, sem)
    gcp.start()
    gcp.wait()
    pltpu.sync_copy(rows_v, pos_hbm)


def _sc_pos_table(arh16, arw16, emb2):
  mesh = plsc.VectorSubcoreMesh(core_axis_name="c", subcore_axis_name="s")
  run = functools.partial(
      pl.kernel,
      out_type=jax.ShapeDtypeStruct((NSLAB, DIM), jnp.float32),
      mesh=mesh,
      scratch_types=[
          pltpu.VMEM((16,), jnp.int32),
          pltpu.VMEM((16,), jnp.int32),
          pltpu.VMEM((16,), jnp.int32),
          pltpu.VMEM((NSLAB, DIM), jnp.float32),
          pltpu.SemaphoreType.DMA,
      ],
  )(_pos_body)
  return run(arh16, arw16, emb2)


def _add_body(x_ref, pos_ref, gate_ref, o_ref):
  g = jnp.tanh(gate_ref[0, 0])
  o_ref[...] = x_ref[...] + pos_ref[...] * g


def _tc_add(xt, pos4, gate2):
  grid = (BSZ, pl.cdiv(NTOK, TB))
  return pl.pallas_call(
      _add_body,
      grid=grid,
      in_specs=[
          pl.BlockSpec((1, TB, NTILE, DIM), lambda b, j: (b, j, 0, 0)),
          pl.BlockSpec((1, 1, NTILE, DIM), lambda b, j: (b, 0, 0, 0)),
          pl.BlockSpec(memory_space=pltpu.SMEM),
      ],
      out_specs=pl.BlockSpec((1, TB, NTILE, DIM), lambda b, j: (b, j, 0, 0)),
      out_shape=jax.ShapeDtypeStruct((BSZ, NTOK, NTILE, DIM), jnp.float32),
  )(xt, pos4, gate2)


@jax.jit
def kernel(x, aspect_ratio, embedding, gate):
  ar32 = aspect_ratio.astype(jnp.int32)
  arh16 = jnp.repeat(ar32[:, 0], NTILE)
  arw16 = jnp.repeat(ar32[:, 1], NTILE)
  gate2 = gate.astype(jnp.float32).reshape(1, 1)
  emb2 = jnp.concatenate(
      [embedding.reshape(NSLAB, DIM),
       jnp.zeros((1, DIM), jnp.float32)])

  pos = _sc_pos_table(arh16, arw16, emb2)
  xt = x.transpose(0, 2, 1, 3)
  out = _tc_add(xt, pos.reshape(BSZ, 1, NTILE, DIM), gate2)
  return out.transpose(0, 2, 1, 3)


# speedup vs baseline: 13.4165x; 1.0066x over previous
import functools

import jax
import jax.numpy as jnp
from jax import lax
from jax.experimental import pallas as pl
from jax.experimental.pallas import tpu as pltpu
from jax.experimental.pallas import tpu_sc as plsc

BSZ = 4
NTILE = 4
NTOK = 1601
DIM = 1280
NSLAB = BSZ * NTILE
TB = 401


def _pos_body(ar_hbm, emb_hbm, pos_hbm, ar_v, idx_v, rows_v, sem):
  cid = lax.axis_index("c")
  sid = lax.axis_index("s")
  wid = sid * 2 + cid

  @pl.when(wid == 0)
  def _():
    pltpu.sync_copy(ar_hbm, ar_v)

    lanes = lax.iota(jnp.int32, 16)
    four = jnp.full((16,), NTILE, jnp.int32)
    one16 = jnp.full((16,), 1, jnp.int32)
    t_vec = lax.rem(lanes, four)
    h_vec = ar_v[pl.ds(0, 16)]
    w_vec = ar_v[pl.ds(16, 16)]
    w_safe = jnp.maximum(w_vec, one16)
    row = lax.div(t_vec, w_safe)
    col = lax.rem(t_vec, w_safe)
    m = jnp.minimum(jnp.maximum(h_vec * w_vec - t_vec, one16 - one16), one16)
    emb_idx = m * (row * four + col) + (one16 - m) * jnp.full(
        (16,), NSLAB, jnp.int32)
    idx_v[...] = emb_idx

    gcp = pltpu.make_async_copy(emb_hbm.at[idx_v], rows_v, sem)
    gcp.start()
    gcp.wait()
    pltpu.sync_copy(rows_v, pos_hbm)


def _sc_pos_table(ar32x, emb2):
  mesh = plsc.VectorSubcoreMesh(core_axis_name="c", subcore_axis_name="s")
  run = functools.partial(
      pl.kernel,
      out_type=jax.ShapeDtypeStruct((NSLAB, DIM), jnp.float32),
      mesh=mesh,
      scratch_types=[
          pltpu.VMEM((32,), jnp.int32),
          pltpu.VMEM((16,), jnp.int32),
          pltpu.VMEM((NSLAB, DIM), jnp.float32),
          pltpu.SemaphoreType.DMA,
      ],
  )(_pos_body)
  return run(ar32x, emb2)


def _add_body(x_ref, pos_ref, gate_ref, o_ref):
  g = jnp.tanh(gate_ref[0, 0])
  o_ref[...] = x_ref[...] + pos_ref[...] * g


def _tc_add(xt, pos4, gate2):
  grid = (BSZ, pl.cdiv(NTOK, TB))
  return pl.pallas_call(
      _add_body,
      grid=grid,
      in_specs=[
          pl.BlockSpec((1, TB, NTILE, DIM), lambda b, j: (b, j, 0, 0)),
          pl.BlockSpec((1, 1, NTILE, DIM), lambda b, j: (b, 0, 0, 0)),
          pl.BlockSpec(memory_space=pltpu.SMEM),
      ],
      out_specs=pl.BlockSpec((1, TB, NTILE, DIM), lambda b, j: (b, j, 0, 0)),
      out_shape=jax.ShapeDtypeStruct((BSZ, NTOK, NTILE, DIM), jnp.float32),
  )(xt, pos4, gate2)


@jax.jit
def kernel(x, aspect_ratio, embedding, gate):
  ar32 = aspect_ratio.astype(jnp.int32)
  ar32x = jnp.concatenate(
      [jnp.repeat(ar32[:, 0], NTILE), jnp.repeat(ar32[:, 1], NTILE)])
  gate2 = gate.astype(jnp.float32).reshape(1, 1)
  emb2 = jnp.concatenate(
      [embedding.reshape(NSLAB, DIM),
       jnp.zeros((1, DIM), jnp.float32)])

  pos = _sc_pos_table(ar32x, emb2)
  xt = x.transpose(0, 2, 1, 3)
  out = _tc_add(xt, pos.reshape(BSZ, 1, NTILE, DIM), gate2)
  return out.transpose(0, 2, 1, 3)


# speedup vs baseline: 13.6469x; 1.0172x over previous
import functools

import jax
import jax.numpy as jnp
from jax import lax
from jax.experimental import pallas as pl
from jax.experimental.pallas import tpu as pltpu
from jax.experimental.pallas import tpu_sc as plsc

BSZ = 4
NTILE = 4
NTOK = 1601
DIM = 1280
NSLAB = BSZ * NTILE
TB = 401


def _pos_body(ar_hbm, emb_hbm, pos_hbm, ar_v, idx_v, rows_v, sem):
  pltpu.sync_copy(ar_hbm, ar_v)

  lanes = lax.iota(jnp.int32, 16)
  four = jnp.full((16,), NTILE, jnp.int32)
  one16 = jnp.full((16,), 1, jnp.int32)
  t_vec = lax.rem(lanes, four)
  h_vec = ar_v[pl.ds(0, 16)]
  w_vec = ar_v[pl.ds(16, 16)]
  w_safe = jnp.maximum(w_vec, one16)
  row = lax.div(t_vec, w_safe)
  col = lax.rem(t_vec, w_safe)
  m = jnp.minimum(jnp.maximum(h_vec * w_vec - t_vec, one16 - one16), one16)
  emb_idx = m * (row * four + col) + (one16 - m) * jnp.full(
      (16,), NSLAB, jnp.int32)
  idx_v[...] = emb_idx

  gcp = pltpu.make_async_copy(emb_hbm.at[idx_v], rows_v, sem)
  gcp.start()
  gcp.wait()
  pltpu.sync_copy(rows_v, pos_hbm)


def _sc_pos_table(ar32x, emb2):
  mesh = plsc.VectorSubcoreMesh(core_axis_name="c", subcore_axis_name="s",
                                num_cores=1, num_subcores=1)
  run = functools.partial(
      pl.kernel,
      out_type=jax.ShapeDtypeStruct((NSLAB, DIM), jnp.float32),
      mesh=mesh,
      scratch_types=[
          pltpu.VMEM((32,), jnp.int32),
          pltpu.VMEM((16,), jnp.int32),
          pltpu.VMEM((NSLAB, DIM), jnp.float32),
          pltpu.SemaphoreType.DMA,
      ],
  )(_pos_body)
  return run(ar32x, emb2)


def _add_body(x_ref, pos_ref, gate_ref, o_ref):
  g = jnp.tanh(gate_ref[0, 0])
  o_ref[...] = x_ref[...] + pos_ref[...] * g


def _tc_add(xt, pos4, gate2):
  grid = (BSZ, pl.cdiv(NTOK, TB))
  return pl.pallas_call(
      _add_body,
      grid=grid,
      in_specs=[
          pl.BlockSpec((1, TB, NTILE, DIM), lambda b, j: (b, j, 0, 0)),
          pl.BlockSpec((1, 1, NTILE, DIM), lambda b, j: (b, 0, 0, 0)),
          pl.BlockSpec(memory_space=pltpu.SMEM),
      ],
      out_specs=pl.BlockSpec((1, TB, NTILE, DIM), lambda b, j: (b, j, 0, 0)),
      out_shape=jax.ShapeDtypeStruct((BSZ, NTOK, NTILE, DIM), jnp.float32),
  )(xt, pos4, gate2)


@jax.jit
def kernel(x, aspect_ratio, embedding, gate):
  ar32 = aspect_ratio.astype(jnp.int32)
  ar32x = jnp.concatenate(
      [jnp.repeat(ar32[:, 0], NTILE), jnp.repeat(ar32[:, 1], NTILE)])
  gate2 = gate.astype(jnp.float32).reshape(1, 1)
  emb2 = jnp.concatenate(
      [embedding.reshape(NSLAB, DIM),
       jnp.zeros((1, DIM), jnp.float32)])

  pos = _sc_pos_table(ar32x, emb2)
  xt = x.transpose(0, 2, 1, 3)
  out = _tc_add(xt, pos.reshape(BSZ, 1, NTILE, DIM), gate2)
  return out.transpose(0, 2, 1, 3)


# speedup vs baseline: 13.6863x; 1.0029x over previous
import functools

import jax
import jax.numpy as jnp
from jax import lax
from jax.experimental import pallas as pl
from jax.experimental.pallas import tpu as pltpu
from jax.experimental.pallas import tpu_sc as plsc

BSZ = 4
NTILE = 4
NTOK = 1601
DIM = 1280
NSLAB = BSZ * NTILE
TB = 534


def _pos_body(ar_hbm, emb_hbm, pos_hbm, ar_v, idx_v, rows_v, sem):
  pltpu.sync_copy(ar_hbm, ar_v)

  lanes = lax.iota(jnp.int32, 16)
  four = jnp.full((16,), NTILE, jnp.int32)
  one16 = jnp.full((16,), 1, jnp.int32)
  t_vec = lax.rem(lanes, four)
  h_vec = ar_v[pl.ds(0, 16)]
  w_vec = ar_v[pl.ds(16, 16)]
  w_safe = jnp.maximum(w_vec, one16)
  row = lax.div(t_vec, w_safe)
  col = lax.rem(t_vec, w_safe)
  m = jnp.minimum(jnp.maximum(h_vec * w_vec - t_vec, one16 - one16), one16)
  emb_idx = m * (row * four + col) + (one16 - m) * jnp.full(
      (16,), NSLAB, jnp.int32)
  idx_v[...] = emb_idx

  gcp = pltpu.make_async_copy(emb_hbm.at[idx_v], rows_v, sem)
  gcp.start()
  gcp.wait()
  pltpu.sync_copy(rows_v, pos_hbm)


def _sc_pos_table(ar32x, emb2):
  mesh = plsc.VectorSubcoreMesh(core_axis_name="c", subcore_axis_name="s",
                                num_cores=1, num_subcores=1)
  run = functools.partial(
      pl.kernel,
      out_type=jax.ShapeDtypeStruct((NSLAB, DIM), jnp.float32),
      mesh=mesh,
      scratch_types=[
          pltpu.VMEM((32,), jnp.int32),
          pltpu.VMEM((16,), jnp.int32),
          pltpu.VMEM((NSLAB, DIM), jnp.float32),
          pltpu.SemaphoreType.DMA,
      ],
  )(_pos_body)
  return run(ar32x, emb2)


def _add_body(x_ref, pos_ref, gate_ref, o_ref):
  g = jnp.tanh(gate_ref[0, 0])
  o_ref[...] = x_ref[...] + pos_ref[...] * g


def _tc_add(xt, pos4, gate2):
  grid = (BSZ, pl.cdiv(NTOK, TB))
  return pl.pallas_call(
      _add_body,
      grid=grid,
      in_specs=[
          pl.BlockSpec((1, TB, NTILE, DIM), lambda b, j: (b, j, 0, 0)),
          pl.BlockSpec((1, 1, NTILE, DIM), lambda b, j: (b, 0, 0, 0)),
          pl.BlockSpec(memory_space=pltpu.SMEM),
      ],
      out_specs=pl.BlockSpec((1, TB, NTILE, DIM), lambda b, j: (b, j, 0, 0)),
      out_shape=jax.ShapeDtypeStruct((BSZ, NTOK, NTILE, DIM), jnp.float32),
  )(xt, pos4, gate2)


@jax.jit
def kernel(x, aspect_ratio, embedding, gate):
  ar32 = aspect_ratio.astype(jnp.int32)
  ar32x = jnp.concatenate(
      [jnp.repeat(ar32[:, 0], NTILE), jnp.repeat(ar32[:, 1], NTILE)])
  gate2 = gate.astype(jnp.float32).reshape(1, 1)
  emb2 = jnp.concatenate(
      [embedding.reshape(NSLAB, DIM),
       jnp.zeros((1, DIM), jnp.float32)])

  pos = _sc_pos_table(ar32x, emb2)
  xt = x.transpose(0, 2, 1, 3)
  out = _tc_add(xt, pos.reshape(BSZ, 1, NTILE, DIM), gate2)
  return out.transpose(0, 2, 1, 3)


# speedup vs baseline: 13.9215x; 1.0172x over previous
import functools

import jax
import jax.numpy as jnp
from jax import lax
from jax.experimental import pallas as pl
from jax.experimental.pallas import tpu as pltpu
from jax.experimental.pallas import tpu_sc as plsc

BSZ = 4
NTILE = 4
NTOK = 1601
DIM = 1280
NSLAB = BSZ * NTILE
TB = 534


def _pos_body(ar_hbm, emb_hbm, pos_hbm, ar_v, idx_v, rows_v, sem):
  pltpu.sync_copy(ar_hbm, ar_v)

  lanes = lax.iota(jnp.int32, 16)
  four = jnp.full((16,), NTILE, jnp.int32)
  one16 = jnp.full((16,), 1, jnp.int32)
  t_vec = lax.rem(lanes, four)
  h_vec = ar_v[pl.ds(0, 16)]
  w_vec = ar_v[pl.ds(16, 16)]
  w_safe = jnp.maximum(w_vec, one16)
  row = lax.div(t_vec, w_safe)
  col = lax.rem(t_vec, w_safe)
  m = jnp.minimum(jnp.maximum(h_vec * w_vec - t_vec, one16 - one16), one16)
  emb_idx = m * (row * four + col) + (one16 - m) * jnp.full(
      (16,), NSLAB, jnp.int32)
  idx_v[...] = emb_idx

  gcp = pltpu.make_async_copy(emb_hbm.at[idx_v], rows_v, sem)
  gcp.start()
  gcp.wait()
  cps = [pltpu.make_async_copy(rows_v.at[pl.ds(b * NTILE, NTILE)],
                               pos_hbm.at[b], sem) for b in range(BSZ)]
  for cp in cps:
    cp.start()
  for cp in cps:
    cp.wait()


def _sc_pos_table(ar32x, emb2):
  mesh = plsc.VectorSubcoreMesh(core_axis_name="c", subcore_axis_name="s",
                                num_cores=1, num_subcores=1)
  run = functools.partial(
      pl.kernel,
      out_type=jax.ShapeDtypeStruct((BSZ, NTILE, DIM), jnp.float32),
      mesh=mesh,
      scratch_types=[
          pltpu.VMEM((32,), jnp.int32),
          pltpu.VMEM((16,), jnp.int32),
          pltpu.VMEM((NSLAB, DIM), jnp.float32),
          pltpu.SemaphoreType.DMA,
      ],
  )(_pos_body)
  return run(ar32x, emb2)


def _add_body(x_ref, pos_ref, gate_ref, o_ref):
  g = jnp.tanh(gate_ref[0, 0])
  o_ref[...] = x_ref[...] + pos_ref[...][:, None] * g


def _tc_add(xt, pos4, gate2):
  grid = (BSZ, pl.cdiv(NTOK, TB))
  return pl.pallas_call(
      _add_body,
      grid=grid,
      in_specs=[
          pl.BlockSpec((1, TB, NTILE, DIM), lambda b, j: (b, j, 0, 0)),
          pl.BlockSpec((1, NTILE, DIM), lambda b, j: (b, 0, 0)),
          pl.BlockSpec(memory_space=pltpu.SMEM),
      ],
      out_specs=pl.BlockSpec((1, TB, NTILE, DIM), lambda b, j: (b, j, 0, 0)),
      out_shape=jax.ShapeDtypeStruct((BSZ, NTOK, NTILE, DIM), jnp.float32),
  )(xt, pos4, gate2)


@jax.jit
def kernel(x, aspect_ratio, embedding, gate):
  ar32 = aspect_ratio.astype(jnp.int32)
  ar32x = jnp.broadcast_to(ar32.T.reshape(8)[:, None], (8, NTILE)).reshape(32)
  gate2 = gate.astype(jnp.float32).reshape(1, 1)
  emb2 = jnp.concatenate(
      [embedding.reshape(NSLAB, DIM),
       jnp.zeros((1, DIM), jnp.float32)])

  pos = _sc_pos_table(ar32x, emb2)
  xt = x.transpose(0, 2, 1, 3)
  out = _tc_add(xt, pos, gate2)
  return out.transpose(0, 2, 1, 3)


# speedup vs baseline: 13.9301x; 1.0006x over previous
import functools

import jax
import jax.numpy as jnp
from jax import lax
from jax.experimental import pallas as pl
from jax.experimental.pallas import tpu as pltpu
from jax.experimental.pallas import tpu_sc as plsc

BSZ = 4
NTILE = 4
NTOK = 1601
DIM = 1280
NSLAB = BSZ * NTILE
TB = 534


def _pos_body(ar_hbm, emb_hbm, pos_hbm, ar_v, idx_v, rows_v, sem):
  pltpu.sync_copy(ar_hbm, ar_v)

  lanes = lax.iota(jnp.int32, 16)
  four = jnp.full((16,), NTILE, jnp.int32)
  one16 = jnp.full((16,), 1, jnp.int32)
  t_vec = lax.rem(lanes, four)
  h_vec = ar_v[pl.ds(0, 16)]
  w_vec = ar_v[pl.ds(16, 16)]
  w_safe = jnp.maximum(w_vec, one16)
  row = lax.div(t_vec, w_safe)
  col = lax.rem(t_vec, w_safe)
  m = jnp.minimum(jnp.maximum(h_vec * w_vec - t_vec, one16 - one16), one16)
  emb_idx = m * (row * four + col) + (one16 - m) * jnp.full(
      (16,), NSLAB, jnp.int32)
  idx_v[...] = emb_idx

  gcp = pltpu.make_async_copy(emb_hbm.at[idx_v], rows_v, sem)
  gcp.start()
  gcp.wait()
  cps = [pltpu.make_async_copy(rows_v.at[pl.ds(b * NTILE, NTILE)],
                               pos_hbm.at[b], sem) for b in range(BSZ)]
  for cp in cps:
    cp.start()
  for cp in cps:
    cp.wait()


def _sc_pos_table(ar32x, emb2):
  mesh = plsc.VectorSubcoreMesh(core_axis_name="c", subcore_axis_name="s",
                                num_cores=1, num_subcores=1)
  run = functools.partial(
      pl.kernel,
      out_type=jax.ShapeDtypeStruct((BSZ, NTILE, DIM), jnp.float32),
      mesh=mesh,
      compiler_params=pltpu.CompilerParams(skip_device_barrier=True),
      scratch_types=[
          pltpu.VMEM((32,), jnp.int32),
          pltpu.VMEM((16,), jnp.int32),
          pltpu.VMEM((NSLAB, DIM), jnp.float32),
          pltpu.SemaphoreType.DMA,
      ],
  )(_pos_body)
  return run(ar32x, emb2)


def _add_body(x_ref, pos_ref, gate_ref, o_ref):
  g = jnp.tanh(gate_ref[0, 0])
  o_ref[...] = x_ref[...] + pos_ref[...][:, None] * g


def _tc_add(xt, pos4, gate2):
  grid = (BSZ, pl.cdiv(NTOK, TB))
  return pl.pallas_call(
      _add_body,
      grid=grid,
      in_specs=[
          pl.BlockSpec((1, TB, NTILE, DIM), lambda b, j: (b, j, 0, 0)),
          pl.BlockSpec((1, NTILE, DIM), lambda b, j: (b, 0, 0)),
          pl.BlockSpec(memory_space=pltpu.SMEM),
      ],
      out_specs=pl.BlockSpec((1, TB, NTILE, DIM), lambda b, j: (b, j, 0, 0)),
      out_shape=jax.ShapeDtypeStruct((BSZ, NTOK, NTILE, DIM), jnp.float32),
  )(xt, pos4, gate2)


@jax.jit
def kernel(x, aspect_ratio, embedding, gate):
  ar32 = aspect_ratio.astype(jnp.int32)
  ar32x = jnp.broadcast_to(ar32.T.reshape(8)[:, None], (8, NTILE)).reshape(32)
  gate2 = gate.astype(jnp.float32).reshape(1, 1)
  emb2 = jnp.concatenate(
      [embedding.reshape(NSLAB, DIM),
       jnp.zeros((1, DIM), jnp.float32)])

  pos = _sc_pos_table(ar32x, emb2)
  xt = x.transpose(0, 2, 1, 3)
  out = _tc_add(xt, pos, gate2)
  return out.transpose(0, 2, 1, 3)
